# baseline (speedup 1.0000x reference)
import jax
import jax.numpy as jnp
from jax import lax
from jax.experimental import pallas as pl
from jax.experimental.pallas import tpu as pltpu
from jax.experimental.pallas import tpu_sc as plsc

_R = 32
_NW = 32
_NC = 2
_NTILES = 4
_NBT = 32
_EMB = 1280
_NTOK = 1025
_LAST = _R * _NW
_SLOT_LG = 0
_SLOT_FLAG = 1


def _slot(tab, s):
    return tab[s // 8, pl.ds((s % 8) * 16, 16)]


def _compute_plain(xb, lb, lgv, n_rows, off=0):
    def step(j, _):
        r = off + j // 10
        c0 = (j % 10) * 128
        for u in range(8):
            sl = pl.ds(c0 + u * 16, 16)
            xb[r, sl] = xb[r, sl] + lb[r, sl] * lgv
        return 0
    lax.fori_loop(0, n_rows * 10, step, 0)


def _compute_gated(xb, lb, gb, lgv, cv, n_rows):
    def step(j, _):
        r = j // 10
        c0 = (j % 10) * 128
        for u in range(8):
            sl = pl.ds(c0 + u * 16, 16)
            xb[r, sl] = xb[r, sl] + lb[r, sl] * lgv + gb[r, sl] * cv
        return 0
    lax.fori_loop(0, n_rows * 10, step, 0)


def _sc_body(x_hbm, lpe_hbm, gpe_hbm, tabs_hbm, coefs_hbm, sels_hbm, out_hbm,
             xbuf0, xbuf1, lbuf, tab, cfbuf, selvbuf,
             si0, si1, si0b, si1b, so0, so1, sg):
    wid = lax.axis_index("s") * _NC + lax.axis_index("c")
    r0 = wid * _R

    xbufs = (xbuf0, xbuf1)
    sin = (si0, si1)
    sinb = (si0b, si1b)
    sout = (so0, so1)

    pltpu.sync_copy(lpe_hbm.at[pl.ds(r0, _R)], lbuf)
    pltpu.sync_copy(tabs_hbm, tab)

    lgv = _slot(tab, _SLOT_LG)
    anyg = _slot(tab, _SLOT_FLAG)[0] > 0.0

    def bt(i):
        return i // _NTILES, lax.rem(i, _NTILES)

    def start_in(i, k):
        b, t = bt(i)
        pltpu.async_copy(x_hbm.at[b, t, pl.ds(r0, 16)],
                         xbufs[k].at[pl.ds(0, 16)], sin[k])
        pltpu.async_copy(x_hbm.at[b, t, pl.ds(r0 + 16, 16)],
                         xbufs[k].at[pl.ds(16, 16)], sinb[k])

    def wait_in_half(k, h):
        b, t = bt(0)
        sem = sin[k] if h == 0 else sinb[k]
        pltpu.make_async_copy(x_hbm.at[b, t, pl.ds(r0, 16)],
                              xbufs[k].at[pl.ds(0, 16)], sem).wait()

    def start_out(i, k):
        b, t = bt(i)
        pltpu.async_copy(xbufs[k], out_hbm.at[b, t, pl.ds(r0, _R)], sout[k])

    def start_out_half(i, k, h):
        b, t = bt(i)
        pltpu.async_copy(xbufs[k].at[pl.ds(h * 16, 16)],
                         out_hbm.at[b, t, pl.ds(r0 + h * 16, 16)], sout[k])

    def wait_out_half(k):
        b, t = bt(0)
        pltpu.make_async_copy(xbufs[k].at[pl.ds(0, 16)],
                              out_hbm.at[b, t, pl.ds(r0, 16)], sout[k]).wait()

    def wait_out(k):
        b, t = bt(0)
        pltpu.make_async_copy(xbufs[k], out_hbm.at[b, t, pl.ds(r0, _R)], sout[k]).wait()

    @pl.when(jnp.logical_not(anyg))
    def _fast():
        start_in(0, 0)
        start_in(1, 1)
        for i in range(_NBT):
            k = i % 2
            wait_in_half(k, 0)
            _compute_plain(xbufs[k], lbuf, lgv, 16, off=0)
            start_out_half(i, k, 0)
            wait_in_half(k, 1)
            _compute_plain(xbufs[k], lbuf, lgv, 16, off=16)
            start_out_half(i, k, 1)
            if i + 2 < _NBT:
                wait_out_half(k)
                wait_out_half(k)
                start_in(i + 2, k)
        wait_out_half(0)
        wait_out_half(0)
        wait_out_half(1)
        wait_out_half(1)

    @pl.when(anyg)
    def _gated():

        def tile_step(i, _):
            b, t = bt(i)
            pltpu.sync_copy(coefs_hbm.at[i], cfbuf)
            pltpu.sync_copy(x_hbm.at[b, t, pl.ds(r0, _R)], xbuf0)
            cv = cfbuf[...]
            tile_on = jnp.abs(cv[0]) > 0.0

            @pl.when(tile_on)
            def _with_gpe():
                pltpu.sync_copy(sels_hbm.at[i], selvbuf)
                sel0 = selvbuf[...][0]
                srow = sel0 // _NTILES
                scol = lax.rem(sel0, _NTILES)
                pltpu.sync_copy(gpe_hbm.at[srow, scol, pl.ds(r0, _R)], xbuf1)
                _compute_gated(xbuf0, lbuf, xbuf1, lgv, cv, _R)

            @pl.when(jnp.logical_not(tile_on))
            def _plain():
                _compute_plain(xbuf0, lbuf, lgv, _R)

            pltpu.sync_copy(xbuf0, out_hbm.at[b, t, pl.ds(r0, _R)])
            return 0

        lax.fori_loop(0, _NBT, tile_step, 0)

    b, t = bt(wid)
    pltpu.sync_copy(x_hbm.at[b, t, pl.ds(_LAST, 1)], xbuf0.at[pl.ds(0, 1)])
    pltpu.sync_copy(lpe_hbm.at[pl.ds(_LAST, 1)], lbuf.at[pl.ds(0, 1)])
    pltpu.sync_copy(coefs_hbm.at[wid], cfbuf)
    cv = cfbuf[...]
    tile_on = jnp.abs(cv[0]) > 0.0

    @pl.when(tile_on)
    def _ep_gated():
        pltpu.sync_copy(sels_hbm.at[wid], selvbuf)
        sel0 = selvbuf[...][0]
        srow = sel0 // _NTILES
        scol = lax.rem(sel0, _NTILES)
        pltpu.sync_copy(gpe_hbm.at[srow, scol, pl.ds(_LAST, 1)], xbuf1.at[pl.ds(0, 1)])
        _compute_gated(xbuf0, lbuf, xbuf1, lgv, cv, 1)

    @pl.when(jnp.logical_not(tile_on))
    def _ep_plain():
        _compute_plain(xbuf0, lbuf, lgv, 1)

    pltpu.sync_copy(xbuf0.at[pl.ds(0, 1)], out_hbm.at[b, t, pl.ds(_LAST, 1)])


def kernel(x, aspect_ratio, local_pe, global_pe, gate):
    bsz, n_tiles, n_tokens, embed_dim = x.shape

    g = jnp.tanh(gate)[0]
    t = jnp.arange(n_tiles, dtype=jnp.int32)
    h = aspect_ratio[:, 0:1]
    w = aspect_ratio[:, 1:2]
    w_safe = jnp.maximum(w, 1)
    row = (t[None, :] // w_safe).astype(jnp.int32)
    col = (t[None, :] % w_safe).astype(jnp.int32)
    mask = t[None, :] < (h * w)
    c = jnp.where(mask.reshape(-1), g, jnp.float32(0.0))
    sel = jnp.where(mask, row * n_tiles + col, 0).reshape(-1)

    entries = jnp.stack([1.0 - g, jnp.max(jnp.abs(c))])
    tabs = jnp.zeros((16 * 128,), jnp.float32)
    tabs = tabs.at[:32].set(jnp.repeat(entries, 16)).reshape(16, 128)

    coefs_v = jnp.broadcast_to(c[:, None], (bsz * n_tiles, 16))
    sels_v = jnp.broadcast_to(sel[:, None].astype(jnp.int32),
                              (bsz * n_tiles, 16))
    mesh = plsc.VectorSubcoreMesh(core_axis_name="c", subcore_axis_name="s")
    f = pl.kernel(
        _sc_body,
        out_type=jax.ShapeDtypeStruct(x.shape, x.dtype),
        mesh=mesh,
        scratch_types=[
            pltpu.VMEM((_R, _EMB), jnp.float32),
            pltpu.VMEM((_R, _EMB), jnp.float32),
            pltpu.VMEM((_R, _EMB), jnp.float32),
            pltpu.VMEM((16, 128), jnp.float32),
            pltpu.VMEM((16,), jnp.float32),
            pltpu.VMEM((16,), jnp.int32),
            pltpu.SemaphoreType.DMA,
            pltpu.SemaphoreType.DMA,
            pltpu.SemaphoreType.DMA,
            pltpu.SemaphoreType.DMA,
            pltpu.SemaphoreType.DMA,
            pltpu.SemaphoreType.DMA,
            pltpu.SemaphoreType.DMA,
        ],
    )
    return f(x, local_pe, global_pe, tabs, coefs_v, sels_v)


# speedup vs baseline: 1.0006x; 1.0006x over previous
import jax
import jax.numpy as jnp
from jax import lax
from jax.experimental import pallas as pl
from jax.experimental.pallas import tpu as pltpu
from jax.experimental.pallas import tpu_sc as plsc

_R = 32
_NW = 32
_NC = 2
_NTILES = 4
_NBT = 32
_EMB = 1280
_NTOK = 1025
_LAST = _R * _NW
_SLOT_LG = 0
_SLOT_FLAG = 1


def _slot(tab, s):
    return tab[s // 8, pl.ds((s % 8) * 16, 16)]


def _compute_plain(xb, lb, lgv, n_rows, off=0):
    def step(j, _):
        r = off + j // 10
        c0 = (j % 10) * 128
        for u in range(8):
            sl = pl.ds(c0 + u * 16, 16)
            xb[r, sl] = xb[r, sl] + lb[r, sl] * lgv
        return 0
    lax.fori_loop(0, n_rows * 10, step, 0)


def _compute_gated(xb, lb, gb, lgv, cv, n_rows):
    def step(j, _):
        r = j // 10
        c0 = (j % 10) * 128
        for u in range(8):
            sl = pl.ds(c0 + u * 16, 16)
            xb[r, sl] = xb[r, sl] + lb[r, sl] * lgv + gb[r, sl] * cv
        return 0
    lax.fori_loop(0, n_rows * 10, step, 0)


def _sc_body(x_hbm, lpe_hbm, gpe_hbm, tabs_hbm, coefs_hbm, sels_hbm, out_hbm,
             xbuf0, xbuf1, lbuf, tab, cfbuf, selvbuf,
             si0, si1, so0, so1, sg):
    wid = lax.axis_index("s") * _NC + lax.axis_index("c")
    r0 = wid * _R

    xbufs = (xbuf0, xbuf1)
    sin = (si0, si1)
    sout = (so0, so1)

    pltpu.sync_copy(lpe_hbm.at[pl.ds(r0, _R)], lbuf)
    pltpu.sync_copy(tabs_hbm, tab)

    lgv = _slot(tab, _SLOT_LG)
    anyg = _slot(tab, _SLOT_FLAG)[0] > 0.0

    def bt(i):
        return i // _NTILES, lax.rem(i, _NTILES)

    def start_in(i, k):
        b, t = bt(i)
        pltpu.async_copy(x_hbm.at[b, t, pl.ds(r0, _R)], xbufs[k], sin[k])

    def wait_in(k):
        b, t = bt(0)
        pltpu.make_async_copy(x_hbm.at[b, t, pl.ds(r0, _R)], xbufs[k], sin[k]).wait()

    def start_out(i, k):
        b, t = bt(i)
        pltpu.async_copy(xbufs[k], out_hbm.at[b, t, pl.ds(r0, _R)], sout[k])

    def start_out_half(i, k, h):
        b, t = bt(i)
        pltpu.async_copy(xbufs[k].at[pl.ds(h * 16, 16)],
                         out_hbm.at[b, t, pl.ds(r0 + h * 16, 16)], sout[k])

    def wait_out_half(k):
        b, t = bt(0)
        pltpu.make_async_copy(xbufs[k].at[pl.ds(0, 16)],
                              out_hbm.at[b, t, pl.ds(r0, 16)], sout[k]).wait()

    def wait_out(k):
        b, t = bt(0)
        pltpu.make_async_copy(xbufs[k], out_hbm.at[b, t, pl.ds(r0, _R)], sout[k]).wait()

    @pl.when(jnp.logical_not(anyg))
    def _fast():
        start_in(0, 0)
        start_in(1, 1)
        for i in range(_NBT):
            k = i % 2
            wait_in(k)
            _compute_plain(xbufs[k], lbuf, lgv, 16, off=0)
            start_out_half(i, k, 0)
            _compute_plain(xbufs[k], lbuf, lgv, 16, off=16)
            start_out_half(i, k, 1)
            if i + 2 < _NBT:
                wait_out_half(k)
                wait_out_half(k)
                start_in(i + 2, k)
        wait_out_half(0)
        wait_out_half(0)
        wait_out_half(1)
        wait_out_half(1)

    @pl.when(anyg)
    def _gated():

        def tile_step(i, _):
            b, t = bt(i)
            pltpu.sync_copy(coefs_hbm.at[i], cfbuf)
            pltpu.sync_copy(x_hbm.at[b, t, pl.ds(r0, _R)], xbuf0)
            cv = cfbuf[...]
            tile_on = jnp.abs(cv[0]) > 0.0

            @pl.when(tile_on)
            def _with_gpe():
                pltpu.sync_copy(sels_hbm.at[i], selvbuf)
                sel0 = selvbuf[...][0]
                srow = sel0 // _NTILES
                scol = lax.rem(sel0, _NTILES)
                pltpu.sync_copy(gpe_hbm.at[srow, scol, pl.ds(r0, _R)], xbuf1)
                _compute_gated(xbuf0, lbuf, xbuf1, lgv, cv, _R)

            @pl.when(jnp.logical_not(tile_on))
            def _plain():
                _compute_plain(xbuf0, lbuf, lgv, _R)

            pltpu.sync_copy(xbuf0, out_hbm.at[b, t, pl.ds(r0, _R)])
            return 0

        lax.fori_loop(0, _NBT, tile_step, 0)

    b, t = bt(wid)
    pltpu.sync_copy(x_hbm.at[b, t, pl.ds(_LAST, 1)], xbuf0.at[pl.ds(0, 1)])
    pltpu.sync_copy(lpe_hbm.at[pl.ds(_LAST, 1)], lbuf.at[pl.ds(0, 1)])
    pltpu.sync_copy(coefs_hbm.at[wid], cfbuf)
    cv = cfbuf[...]
    tile_on = jnp.abs(cv[0]) > 0.0

    @pl.when(tile_on)
    def _ep_gated():
        pltpu.sync_copy(sels_hbm.at[wid], selvbuf)
        sel0 = selvbuf[...][0]
        srow = sel0 // _NTILES
        scol = lax.rem(sel0, _NTILES)
        pltpu.sync_copy(gpe_hbm.at[srow, scol, pl.ds(_LAST, 1)], xbuf1.at[pl.ds(0, 1)])
        _compute_gated(xbuf0, lbuf, xbuf1, lgv, cv, 1)

    @pl.when(jnp.logical_not(tile_on))
    def _ep_plain():
        _compute_plain(xbuf0, lbuf, lgv, 1)

    pltpu.sync_copy(xbuf0.at[pl.ds(0, 1)], out_hbm.at[b, t, pl.ds(_LAST, 1)])


def kernel(x, aspect_ratio, local_pe, global_pe, gate):
    bsz, n_tiles, n_tokens, embed_dim = x.shape

    g = jnp.tanh(gate)[0]
    t = jnp.arange(n_tiles, dtype=jnp.int32)
    h = aspect_ratio[:, 0:1]
    w = aspect_ratio[:, 1:2]
    w_safe = jnp.maximum(w, 1)
    row = (t[None, :] // w_safe).astype(jnp.int32)
    col = (t[None, :] % w_safe).astype(jnp.int32)
    mask = t[None, :] < (h * w)
    c = jnp.where(mask.reshape(-1), g, jnp.float32(0.0))
    sel = jnp.where(mask, row * n_tiles + col, 0).reshape(-1)

    entries = jnp.stack([1.0 - g, jnp.max(jnp.abs(c))])
    tabs = jnp.zeros((16 * 128,), jnp.float32)
    tabs = tabs.at[:32].set(jnp.repeat(entries, 16)).reshape(16, 128)

    coefs_v = jnp.broadcast_to(c[:, None], (bsz * n_tiles, 16))
    sels_v = jnp.broadcast_to(sel[:, None].astype(jnp.int32),
                              (bsz * n_tiles, 16))
    mesh = plsc.VectorSubcoreMesh(core_axis_name="c", subcore_axis_name="s")
    f = pl.kernel(
        _sc_body,
        out_type=jax.ShapeDtypeStruct(x.shape, x.dtype),
        mesh=mesh,
        scratch_types=[
            pltpu.VMEM((_R, _EMB), jnp.float32),
            pltpu.VMEM((_R, _EMB), jnp.float32),
            pltpu.VMEM((_R, _EMB), jnp.float32),
            pltpu.VMEM((16, 128), jnp.float32),
            pltpu.VMEM((16,), jnp.float32),
            pltpu.VMEM((16,), jnp.int32),
            pltpu.SemaphoreType.DMA,
            pltpu.SemaphoreType.DMA,
            pltpu.SemaphoreType.DMA,
            pltpu.SemaphoreType.DMA,
            pltpu.SemaphoreType.DMA,
        ],
    )
    return f(x, local_pe, global_pe, tabs, coefs_v, sels_v)
